# baseline (speedup 1.0000x reference)
import functools

import jax
import jax.numpy as jnp
from jax import lax
from jax.experimental import pallas as pl
from jax.experimental.pallas import tpu as pltpu
from jax.experimental.pallas import tpu_sc as plsc

N = 10000
E = 320000
G = 256
DH = 128
DDEG = 16

NC = 2
NS = 16
NW = NC * NS
CH = 40
SL = 50
ZB = 80
NZB = N // ZB
ZITER = (NZB + NS - 1) // NS


def _block_loop(sid, fn):
    def it(j, c):
        b = sid + j * NS

        @pl.when(b < NZB)
        def _():
            fn(b * ZB)

        return c

    lax.fori_loop(0, ZITER, it, 0)


def _fill_const(ref, rows, width, value):
    vec = jnp.full((16,), value, jnp.float32)

    def body(r, c):
        for j in range(width // 16):
            ref[r, pl.ds(j * 16, 16)] = vec
        return c

    lax.fori_loop(0, rows, body, 0)


UNROLL = 5


def _make_prop(chunks_per_tile, cols_mode, ch=CH, sl=SL, unroll=UNROLL):
    mesh = plsc.VectorSubcoreMesh(core_axis_name="c", subcore_axis_name="s")
    nstrip = chunks_per_tile // sl
    ngrp = sl // unroll

    @functools.partial(
        pl.kernel,
        out_type=jax.ShapeDtypeStruct((NC, N, DH), jnp.float32),
        mesh=mesh,
        scratch_types=(
            [pltpu.VMEM((sl, ch), jnp.int32)] * 2
            + [pltpu.VMEM((ch, DH), jnp.float32)] * unroll
            + [pltpu.VMEM((ZB, DH), jnp.float32)]
            + [pltpu.SemaphoreType.DMA] * (2 * unroll)
            + [pltpu.VMEM_SHARED((N, DH), jnp.float32)]
        ),
    )
    def prop(y_hbm, src_hbm, dst_hbm, out_hbm, *s):
        src_v, dst_v = s[0], s[1]
        bufs = s[2:2 + unroll]
        zero_v = s[2 + unroll]
        gsem = s[3 + unroll:3 + 2 * unroll]
        ssem = s[3 + 2 * unroll:3 + 3 * unroll]
        acc_sh = s[3 + 3 * unroll]
        cid = lax.axis_index("c")
        sid = lax.axis_index("s")
        row = sid if cols_mode else sid * NC + cid
        _fill_const(zero_v, ZB, DH, 0.0)
        _block_loop(sid, lambda r: pltpu.sync_copy(
            zero_v, acc_sh.at[pl.ds(r, ZB)]))
        plsc.subcore_barrier()
        ysrc = y_hbm.at[cid] if cols_mode else y_hbm

        def drain(k):
            pltpu.make_async_copy(bufs[k], acc_sh.at[dst_v.at[0]],
                                  ssem[k]).wait()

        def strip(st, c):
            @pl.when(st > 0)
            def _():
                for k in range(unroll):
                    drain(k)

            pltpu.sync_copy(src_hbm.at[row, st], src_v)
            pltpu.sync_copy(dst_hbm.at[row, st], dst_v)

            def group(g, c2):
                j0 = g * unroll

                cps = []
                for k in range(unroll):
                    @pl.when(g > 0)
                    def _(k=k):
                        drain(k)

                    cps.append(pltpu.async_copy(ysrc.at[src_v.at[j0 + k]],
                                                bufs[k], gsem[k]))
                for k in range(unroll):
                    cps[k].wait()
                    pltpu.async_copy(bufs[k], acc_sh.at[dst_v.at[j0 + k]],
                                     ssem[k], add=True)
                return c2

            lax.fori_loop(0, ngrp, group, 0)
            return c

        lax.fori_loop(0, nstrip, strip, 0)
        for k in range(unroll):
            drain(k)
        plsc.subcore_barrier()
        _block_loop(sid, lambda r: pltpu.sync_copy(
            acc_sh.at[pl.ds(r, ZB)], out_hbm.at[cid, pl.ds(r, ZB)]))

    return prop


def _make_degree():
    mesh = plsc.VectorSubcoreMesh(core_axis_name="c", subcore_axis_name="s")
    dch, dsl = 80, 25
    chunks_per_tile = E // NW // dch
    nstrip = chunks_per_tile // dsl
    ngrp = dsl // UNROLL

    @functools.partial(
        pl.kernel,
        out_type=jax.ShapeDtypeStruct((NC, N, DH), jnp.float32),
        mesh=mesh,
        scratch_types=(
            [pltpu.VMEM((dsl, dch), jnp.int32),
             pltpu.VMEM((dch, DH), jnp.float32),
             pltpu.VMEM((ZB, DH), jnp.float32)]
            + [pltpu.SemaphoreType.DMA] * UNROLL
            + [pltpu.VMEM_SHARED((N, DH), jnp.float32)]
        ),
    )
    def deg(dst_hbm, out_hbm, *s):
        dst_v, ones_v, zero_v = s[0], s[1], s[2]
        ssem = s[3:3 + UNROLL]
        acc_sh = s[3 + UNROLL]
        cid = lax.axis_index("c")
        sid = lax.axis_index("s")
        wid = sid * NC + cid
        _fill_const(ones_v, dch, DH, 1.0)
        _fill_const(zero_v, ZB, DH, 0.0)
        _block_loop(sid, lambda r: pltpu.sync_copy(
            zero_v, acc_sh.at[pl.ds(r, ZB)]))
        plsc.subcore_barrier()

        def drain(k):
            pltpu.make_async_copy(ones_v, acc_sh.at[dst_v.at[0]],
                                  ssem[k]).wait()

        def strip(st, c):
            @pl.when(st > 0)
            def _():
                for k in range(UNROLL):
                    drain(k)

            pltpu.sync_copy(dst_hbm.at[wid, st], dst_v)

            def group(g, c2):
                j0 = g * UNROLL

                for k in range(UNROLL):
                    @pl.when(g > 0)
                    def _(k=k):
                        drain(k)

                    pltpu.async_copy(ones_v, acc_sh.at[dst_v.at[j0 + k]],
                                     ssem[k], add=True)
                return c2

            lax.fori_loop(0, ngrp, group, 0)
            return c

        lax.fori_loop(0, nstrip, strip, 0)
        for k in range(UNROLL):
            drain(k)
        plsc.subcore_barrier()
        _block_loop(sid, lambda r: pltpu.sync_copy(
            acc_sh.at[pl.ds(r, ZB)], out_hbm.at[cid, pl.ds(r, ZB)]))

    return deg


_BR = 2000
_NBLK = N // _BR


def _xw_body(x, W1, xwh):
    xw = jnp.dot(x[...], W1[...], preferred_element_type=jnp.float32)
    xwh[0] = xw[:, :DH]
    xwh[1] = xw[:, DH:]


def _scale_body(degp, xwh, y1h, dinv_ref):
    d = degp[0, :, 0:1] + degp[1, :, 0:1] + 1.0
    di = lax.rsqrt(d)
    y1h[0] = xwh[0] * di
    y1h[1] = xwh[1] * di
    dinv_ref[...] = di


def _conv_body(z1, y1h, dinv, b1, W2p, y2_ref):
    o1 = jnp.concatenate([z1[0] + y1h[0], z1[1] + y1h[1]], axis=1)
    h1 = jnp.maximum(o1 * dinv[...] + b1[...], 0.0)
    q = jnp.dot(h1, W2p[...], preferred_element_type=jnp.float32)
    y2_ref[...] = q * dinv[...]


def _pooltail_body(zp2, y2, dinv, b2, bi, h3p, Wg, W2, b2l, W3, b3, HW1,
                   hb1, HW2bd, hb2, out, g_acc):
    i = pl.program_id(0)

    @pl.when(i < _NBLK)
    def _():
        h2 = (zp2[0] + zp2[1] + y2[...]) * dinv[...] + b2[...]
        seg = bi[0, 0, :][None, :]
        onehot = (lax.broadcasted_iota(jnp.int32, (G, _BR), 0) == seg)
        contrib = jnp.dot(onehot.astype(jnp.float32), h2,
                          preferred_element_type=jnp.float32,
                          precision=lax.Precision.HIGHEST)

        @pl.when(i == 0)
        def _():
            g_acc[...] = contrib

        @pl.when(i > 0)
        def _():
            g_acc[...] = g_acc[...] + contrib

    @pl.when(i == _NBLK)
    def _():
        h3 = jnp.maximum(
            jnp.dot(g_acc[...], Wg[...], preferred_element_type=jnp.float32)
            + h3p[...], 0.0)
        h4 = jnp.maximum(
            jnp.dot(h3, W2[...], preferred_element_type=jnp.float32)
            + b2l[...], 0.0)
        h5 = jnp.maximum(
            jnp.dot(h4, W3[...], preferred_element_type=jnp.float32)
            + b3[...], 0.0)
        t = jnp.maximum(
            jnp.dot(h5, HW1[...], preferred_element_type=jnp.float32)
            + hb1[...], 0.0)
        out[...] = (jnp.dot(t, HW2bd[...], preferred_element_type=jnp.float32)
                    + hb2[...])


def _lin1pre_body(sv, fp, Ws, Wf, b1, h3p):
    acc = jnp.dot(sv[...], Ws[...], preferred_element_type=jnp.float32)
    acc = acc + jnp.dot(fp[...], Wf[...], preferred_element_type=jnp.float32)
    h3p[...] = acc + b1[...]


def _full(shape):
    return pl.BlockSpec(shape, lambda i: tuple(0 for _ in shape))


def kernel(x, edge_index, edge_attr, batch_index, solvent_descriptors,
           mol_fingerprints, num_graphs, conv1_W, conv1_b, conv2_W, conv2_b,
           lin1_W, lin1_b, lin2_W, lin2_b, lin3_W, lin3_b,
           head_W1, head_b1, head_W2, head_b2):
    src = edge_index[0]
    dst = edge_index[1]
    nblk = N // _BR

    xwh = pl.pallas_call(
        _xw_body,
        grid=(nblk,),
        in_specs=[
            pl.BlockSpec((_BR, 131), lambda i: (i, 0)),
            _full((131, 256)),
        ],
        out_specs=pl.BlockSpec((NC, _BR, DH), lambda i: (0, i, 0)),
        out_shape=jax.ShapeDtypeStruct((NC, N, DH), jnp.float32),
    )(x, conv1_W)

    Ws = lin1_W[64:75]
    Wf = lin1_W[75:]
    CB = 1024
    h3p = pl.pallas_call(
        _lin1pre_body,
        grid=(4096 // CB,),
        in_specs=[
            _full((G, 11)),
            _full((G, 2065)),
            pl.BlockSpec((11, CB), lambda i: (0, i)),
            pl.BlockSpec((2065, CB), lambda i: (0, i)),
            pl.BlockSpec((1, CB), lambda i: (0, i)),
        ],
        out_specs=pl.BlockSpec((G, CB), lambda i: (0, i)),
        out_shape=jax.ShapeDtypeStruct((G, 4096), jnp.float32),
    )(solvent_descriptors, mol_fingerprints, Ws, Wf, lin1_b.reshape(1, 4096))

    degp = _make_degree()(dst.reshape(NW, 5, 25, 80))

    y1h, dinv = pl.pallas_call(
        _scale_body,
        grid=(nblk,),
        in_specs=[
            pl.BlockSpec((NC, _BR, DH), lambda i: (0, i, 0)),
            pl.BlockSpec((NC, _BR, DH), lambda i: (0, i, 0)),
        ],
        out_specs=[
            pl.BlockSpec((NC, _BR, DH), lambda i: (0, i, 0)),
            pl.BlockSpec((_BR, 1), lambda i: (i, 0)),
        ],
        out_shape=[
            jax.ShapeDtypeStruct((NC, N, DH), jnp.float32),
            jax.ShapeDtypeStruct((N, 1), jnp.float32),
        ],
    )(degp, xwh)

    ch1, sl1, u1 = CH, SL, UNROLL
    cpt_c = E // NS // ch1
    srcc = src.reshape(NS, cpt_c // sl1, sl1, ch1)
    dstc = dst.reshape(NS, cpt_c // sl1, sl1, ch1)
    z1 = _make_prop(cpt_c, True, ch1, sl1, u1)(y1h, srcc, dstc)

    W2p = jnp.pad(conv2_W, ((0, 0), (0, DH - conv2_W.shape[1])))
    b1r = conv1_b.reshape(1, 256)
    y2 = pl.pallas_call(
        _conv_body,
        grid=(nblk,),
        in_specs=[
            pl.BlockSpec((NC, _BR, DH), lambda i: (0, i, 0)),
            pl.BlockSpec((NC, _BR, DH), lambda i: (0, i, 0)),
            pl.BlockSpec((_BR, 1), lambda i: (i, 0)),
            _full((1, 256)),
            _full((256, DH)),
        ],
        out_specs=pl.BlockSpec((_BR, DH), lambda i: (i, 0)),
        out_shape=jax.ShapeDtypeStruct((N, DH), jnp.float32),
    )(z1, y1h, dinv, b1r, W2p)

    ch2, sl2, u2 = CH, SL, UNROLL
    cpt_e = E // NW // ch2
    srce = src.reshape(NW, cpt_e // sl2, sl2, ch2)
    dste = dst.reshape(NW, cpt_e // sl2, sl2, ch2)
    z2 = _make_prop(cpt_e, False, ch2, sl2, u2)(y2, srce, dste)

    bi3 = batch_index.reshape(nblk, 1, _BR)
    b2p = jnp.pad(conv2_b, (0, DH - conv2_b.shape[0])).reshape(1, DH)
    Wg = jnp.pad(lin1_W[:64], ((0, DH - 64), (0, 0)))
    HW1 = jnp.transpose(head_W1, (1, 0, 2)).reshape(128, 192)
    hb1 = head_b1.reshape(1, 192)
    HW2bd = jnp.zeros((192, 8), jnp.float32)
    for i in range(6):
        HW2bd = HW2bd.at[i * 32:(i + 1) * 32, i].set(head_W2[i, :, 0])
    hb2 = jnp.pad(head_b2.reshape(1, 6), ((0, 0), (0, 2)))

    def _cap(i):
        return jnp.minimum(i, _NBLK - 1)

    outp = pl.pallas_call(
        _pooltail_body,
        grid=(nblk + 1,),
        in_specs=[
            pl.BlockSpec((NC, _BR, DH), lambda i: (0, _cap(i), 0)),
            pl.BlockSpec((_BR, DH), lambda i: (_cap(i), 0)),
            pl.BlockSpec((_BR, 1), lambda i: (_cap(i), 0)),
            _full((1, DH)),
            pl.BlockSpec((1, 1, _BR), lambda i: (_cap(i), 0, 0)),
            _full((G, 4096)),
            _full((DH, 4096)),
            _full((4096, 512)),
            _full((1, 512)),
            _full((512, 128)),
            _full((1, 128)),
            _full((128, 192)),
            _full((1, 192)),
            _full((192, 8)),
            _full((1, 8)),
        ],
        out_specs=pl.BlockSpec((G, 8), lambda i: (0, 0)),
        out_shape=jax.ShapeDtypeStruct((G, 8), jnp.float32),
        scratch_shapes=[pltpu.VMEM((G, DH), jnp.float32)],
    )(z2, y2, dinv, b2p, bi3, h3p, Wg, lin2_W, lin2_b.reshape(1, 512),
      lin3_W, lin3_b.reshape(1, 128), HW1, hb1, HW2bd, hb2)
    return outp[:, :6]


# speedup vs baseline: 1.0516x; 1.0516x over previous
import functools

import jax
import jax.numpy as jnp
from jax import lax
from jax.experimental import pallas as pl
from jax.experimental.pallas import tpu as pltpu
from jax.experimental.pallas import tpu_sc as plsc

N = 10000
E = 320000
G = 256
DH = 128
DDEG = 16

NC = 2
NS = 16
NW = NC * NS
CH = 40
SL = 50
ZB = 80
NZB = N // ZB
ZITER = (NZB + NS - 1) // NS


def _block_loop(sid, fn, zb=ZB):
    nzb = N // zb
    zit = (nzb + NS - 1) // NS

    def it(j, c):
        b = sid + j * NS

        @pl.when(b < nzb)
        def _():
            fn(b * zb)

        return c

    lax.fori_loop(0, zit, it, 0)


def _fill_const(ref, rows, width, value):
    vec = jnp.full((16,), value, jnp.float32)

    def body(r, c):
        for j in range(width // 16):
            ref[r, pl.ds(j * 16, 16)] = vec
        return c

    lax.fori_loop(0, rows, body, 0)


UNROLL = 5


def _make_prop(chunks_per_tile, cols_mode, ch=CH, sl=SL, unroll=UNROLL):
    mesh = plsc.VectorSubcoreMesh(core_axis_name="c", subcore_axis_name="s")
    nstrip = chunks_per_tile // sl
    ngrp = sl // unroll

    @functools.partial(
        pl.kernel,
        out_type=jax.ShapeDtypeStruct((NC, N, DH), jnp.float32),
        mesh=mesh,
        scratch_types=(
            [pltpu.VMEM((sl, ch), jnp.int32)] * 4
            + [pltpu.VMEM((ch, DH), jnp.float32)] * unroll
            + [pltpu.SemaphoreType.DMA] * (2 * unroll + 2)
            + [pltpu.VMEM_SHARED((N, DH), jnp.float32)]
        ),
    )
    def prop(y_hbm, src_hbm, dst_hbm, out_hbm, *s):
        src_vs, dst_vs = s[0:2], s[2:4]
        bufs = s[4:4 + unroll]
        gsem = s[4 + unroll:4 + 2 * unroll]
        ssem = s[4 + 2 * unroll:4 + 3 * unroll]
        isem = s[4 + 3 * unroll:6 + 3 * unroll]
        acc_sh = s[6 + 3 * unroll]
        cid = lax.axis_index("c")
        sid = lax.axis_index("s")
        row = sid if cols_mode else sid * NC + cid
        i0 = [pltpu.async_copy(src_hbm.at[row, 0], src_vs[0], isem[0]),
              pltpu.async_copy(dst_hbm.at[row, 0], dst_vs[0], isem[1])]
        _fill_const(bufs[0], ch, DH, 0.0)
        _block_loop(sid, lambda r: pltpu.sync_copy(
            bufs[0], acc_sh.at[pl.ds(r, ch)]), ch)
        i0[0].wait()
        i0[1].wait()
        plsc.subcore_barrier()
        ysrc = y_hbm.at[cid] if cols_mode else y_hbm

        def drain(k, dst_v):
            pltpu.make_async_copy(bufs[k], acc_sh.at[dst_v.at[0]],
                                  ssem[k]).wait()

        def idx_wait(p):
            pltpu.make_async_copy(src_hbm.at[row, 0], src_vs[p],
                                  isem[0]).wait()
            pltpu.make_async_copy(dst_hbm.at[row, 0], dst_vs[p],
                                  isem[1]).wait()

        def do_strip(st, p):
            src_v, dst_v = src_vs[p], dst_vs[p]

            @pl.when(st > 0)
            def _():
                idx_wait(p)

            def group(g, c2):
                j0 = g * unroll

                cps = []
                for k in range(unroll):
                    @pl.when((g > 0) | (st > 0))
                    def _(k=k):
                        drain(k, dst_v)

                    cps.append(pltpu.async_copy(ysrc.at[src_v.at[j0 + k]],
                                                bufs[k], gsem[k]))

                @pl.when((g == 0) & (st + 1 < nstrip))
                def _():
                    pltpu.async_copy(src_hbm.at[row, st + 1], src_vs[1 - p],
                                     isem[0])
                    pltpu.async_copy(dst_hbm.at[row, st + 1], dst_vs[1 - p],
                                     isem[1])

                for k in range(unroll):
                    cps[k].wait()
                    pltpu.async_copy(bufs[k], acc_sh.at[dst_v.at[j0 + k]],
                                     ssem[k], add=True)
                return c2

            lax.fori_loop(0, ngrp, group, 0)

        def strip_pair(i, c):
            do_strip(2 * i, 0)
            do_strip(2 * i + 1, 1)
            return c

        lax.fori_loop(0, nstrip // 2, strip_pair, 0)
        for k in range(unroll):
            drain(k, dst_vs[1])
        plsc.subcore_barrier()
        _block_loop(sid, lambda r: pltpu.sync_copy(
            acc_sh.at[pl.ds(r, ZB)], out_hbm.at[cid, pl.ds(r, ZB)]))

    return prop


def _make_degree():
    mesh = plsc.VectorSubcoreMesh(core_axis_name="c", subcore_axis_name="s")
    dch, dsl = 80, 25
    chunks_per_tile = E // NW // dch
    nstrip = chunks_per_tile // dsl
    ngrp = dsl // UNROLL

    @functools.partial(
        pl.kernel,
        out_type=jax.ShapeDtypeStruct((NC, N, DH), jnp.float32),
        mesh=mesh,
        scratch_types=(
            [pltpu.VMEM((dsl, dch), jnp.int32),
             pltpu.VMEM((dch, DH), jnp.float32),
             pltpu.VMEM((ZB, DH), jnp.float32)]
            + [pltpu.SemaphoreType.DMA] * UNROLL
            + [pltpu.VMEM_SHARED((N, DH), jnp.float32)]
        ),
    )
    def deg(dst_hbm, out_hbm, *s):
        dst_v, ones_v, zero_v = s[0], s[1], s[2]
        ssem = s[3:3 + UNROLL]
        acc_sh = s[3 + UNROLL]
        cid = lax.axis_index("c")
        sid = lax.axis_index("s")
        wid = sid * NC + cid
        _fill_const(ones_v, dch, DH, 1.0)
        _fill_const(zero_v, ZB, DH, 0.0)
        _block_loop(sid, lambda r: pltpu.sync_copy(
            zero_v, acc_sh.at[pl.ds(r, ZB)]))
        plsc.subcore_barrier()

        def drain(k):
            pltpu.make_async_copy(ones_v, acc_sh.at[dst_v.at[0]],
                                  ssem[k]).wait()

        def strip(st, c):
            @pl.when(st > 0)
            def _():
                for k in range(UNROLL):
                    drain(k)

            pltpu.sync_copy(dst_hbm.at[wid, st], dst_v)

            def group(g, c2):
                j0 = g * UNROLL

                for k in range(UNROLL):
                    @pl.when(g > 0)
                    def _(k=k):
                        drain(k)

                    pltpu.async_copy(ones_v, acc_sh.at[dst_v.at[j0 + k]],
                                     ssem[k], add=True)
                return c2

            lax.fori_loop(0, ngrp, group, 0)
            return c

        lax.fori_loop(0, nstrip, strip, 0)
        for k in range(UNROLL):
            drain(k)
        plsc.subcore_barrier()
        _block_loop(sid, lambda r: pltpu.sync_copy(
            acc_sh.at[pl.ds(r, ZB)], out_hbm.at[cid, pl.ds(r, ZB)]))

    return deg


_BR = 2000
_NBLK = N // _BR


def _xw_body(x, W1, xwh):
    xw = jnp.dot(x[...], W1[...], preferred_element_type=jnp.float32)
    xwh[0] = xw[:, :DH]
    xwh[1] = xw[:, DH:]


def _scale_body(degp, xwh, y1h, dinv_ref):
    d = degp[0, :, 0:1] + degp[1, :, 0:1] + 1.0
    di = lax.rsqrt(d)
    y1h[0] = xwh[0] * di
    y1h[1] = xwh[1] * di
    dinv_ref[...] = di


def _conv_body(z1, y1h, dinv, b1, W2p, y2_ref):
    o1 = jnp.concatenate([z1[0] + y1h[0], z1[1] + y1h[1]], axis=1)
    h1 = jnp.maximum(o1 * dinv[...] + b1[...], 0.0)
    q = jnp.dot(h1, W2p[...], preferred_element_type=jnp.float32)
    y2_ref[...] = q * dinv[...]


def _pooltail_body(zp2, y2, dinv, b2, bi, h3p, Wg, W2, b2l, W3, b3, HW1,
                   hb1, HW2bd, hb2, out, g_acc):
    i = pl.program_id(0)

    @pl.when(i < _NBLK)
    def _():
        h2 = (zp2[0] + zp2[1] + y2[...]) * dinv[...] + b2[...]
        seg = bi[0, 0, :][None, :]
        onehot = (lax.broadcasted_iota(jnp.int32, (G, _BR), 0) == seg)
        contrib = jnp.dot(onehot.astype(jnp.float32), h2,
                          preferred_element_type=jnp.float32,
                          precision=lax.Precision.HIGHEST)

        @pl.when(i == 0)
        def _():
            g_acc[...] = contrib

        @pl.when(i > 0)
        def _():
            g_acc[...] = g_acc[...] + contrib

    @pl.when(i == _NBLK)
    def _():
        h3 = jnp.maximum(
            jnp.dot(g_acc[...], Wg[...], preferred_element_type=jnp.float32)
            + h3p[...], 0.0)
        h4 = jnp.maximum(
            jnp.dot(h3, W2[...], preferred_element_type=jnp.float32)
            + b2l[...], 0.0)
        h5 = jnp.maximum(
            jnp.dot(h4, W3[...], preferred_element_type=jnp.float32)
            + b3[...], 0.0)
        t = jnp.maximum(
            jnp.dot(h5, HW1[...], preferred_element_type=jnp.float32)
            + hb1[...], 0.0)
        out[...] = (jnp.dot(t, HW2bd[...], preferred_element_type=jnp.float32)
                    + hb2[...])


def _lin1pre_body(sv, fp, Ws, Wf, b1, h3p):
    acc = jnp.dot(sv[...], Ws[...], preferred_element_type=jnp.float32)
    acc = acc + jnp.dot(fp[...], Wf[...], preferred_element_type=jnp.float32)
    h3p[...] = acc + b1[...]


def _full(shape):
    return pl.BlockSpec(shape, lambda i: tuple(0 for _ in shape))


def kernel(x, edge_index, edge_attr, batch_index, solvent_descriptors,
           mol_fingerprints, num_graphs, conv1_W, conv1_b, conv2_W, conv2_b,
           lin1_W, lin1_b, lin2_W, lin2_b, lin3_W, lin3_b,
           head_W1, head_b1, head_W2, head_b2):
    src = edge_index[0]
    dst = edge_index[1]
    nblk = N // _BR

    xwh = pl.pallas_call(
        _xw_body,
        grid=(nblk,),
        in_specs=[
            pl.BlockSpec((_BR, 131), lambda i: (i, 0)),
            _full((131, 256)),
        ],
        out_specs=pl.BlockSpec((NC, _BR, DH), lambda i: (0, i, 0)),
        out_shape=jax.ShapeDtypeStruct((NC, N, DH), jnp.float32),
    )(x, conv1_W)

    Ws = lin1_W[64:75]
    Wf = lin1_W[75:]
    CB = 1024
    h3p = pl.pallas_call(
        _lin1pre_body,
        grid=(4096 // CB,),
        in_specs=[
            _full((G, 11)),
            _full((G, 2065)),
            pl.BlockSpec((11, CB), lambda i: (0, i)),
            pl.BlockSpec((2065, CB), lambda i: (0, i)),
            pl.BlockSpec((1, CB), lambda i: (0, i)),
        ],
        out_specs=pl.BlockSpec((G, CB), lambda i: (0, i)),
        out_shape=jax.ShapeDtypeStruct((G, 4096), jnp.float32),
    )(solvent_descriptors, mol_fingerprints, Ws, Wf, lin1_b.reshape(1, 4096))

    degp = _make_degree()(dst.reshape(NW, 5, 25, 80))

    y1h, dinv = pl.pallas_call(
        _scale_body,
        grid=(nblk,),
        in_specs=[
            pl.BlockSpec((NC, _BR, DH), lambda i: (0, i, 0)),
            pl.BlockSpec((NC, _BR, DH), lambda i: (0, i, 0)),
        ],
        out_specs=[
            pl.BlockSpec((NC, _BR, DH), lambda i: (0, i, 0)),
            pl.BlockSpec((_BR, 1), lambda i: (i, 0)),
        ],
        out_shape=[
            jax.ShapeDtypeStruct((NC, N, DH), jnp.float32),
            jax.ShapeDtypeStruct((N, 1), jnp.float32),
        ],
    )(degp, xwh)

    ch1, sl1, u1 = CH, 25, UNROLL
    cpt_c = E // NS // ch1
    srcc = src.reshape(NS, cpt_c // sl1, sl1, ch1)
    dstc = dst.reshape(NS, cpt_c // sl1, sl1, ch1)
    z1 = _make_prop(cpt_c, True, ch1, sl1, u1)(y1h, srcc, dstc)

    W2p = jnp.pad(conv2_W, ((0, 0), (0, DH - conv2_W.shape[1])))
    b1r = conv1_b.reshape(1, 256)
    y2 = pl.pallas_call(
        _conv_body,
        grid=(nblk,),
        in_specs=[
            pl.BlockSpec((NC, _BR, DH), lambda i: (0, i, 0)),
            pl.BlockSpec((NC, _BR, DH), lambda i: (0, i, 0)),
            pl.BlockSpec((_BR, 1), lambda i: (i, 0)),
            _full((1, 256)),
            _full((256, DH)),
        ],
        out_specs=pl.BlockSpec((_BR, DH), lambda i: (i, 0)),
        out_shape=jax.ShapeDtypeStruct((N, DH), jnp.float32),
    )(z1, y1h, dinv, b1r, W2p)

    ch2, sl2, u2 = CH, 25, UNROLL
    cpt_e = E // NW // ch2
    srce = src.reshape(NW, cpt_e // sl2, sl2, ch2)
    dste = dst.reshape(NW, cpt_e // sl2, sl2, ch2)
    z2 = _make_prop(cpt_e, False, ch2, sl2, u2)(y2, srce, dste)

    bi3 = batch_index.reshape(nblk, 1, _BR)
    b2p = jnp.pad(conv2_b, (0, DH - conv2_b.shape[0])).reshape(1, DH)
    Wg = jnp.pad(lin1_W[:64], ((0, DH - 64), (0, 0)))
    HW1 = jnp.transpose(head_W1, (1, 0, 2)).reshape(128, 192)
    hb1 = head_b1.reshape(1, 192)
    HW2bd = jnp.zeros((192, 8), jnp.float32)
    for i in range(6):
        HW2bd = HW2bd.at[i * 32:(i + 1) * 32, i].set(head_W2[i, :, 0])
    hb2 = jnp.pad(head_b2.reshape(1, 6), ((0, 0), (0, 2)))

    def _cap(i):
        return jnp.minimum(i, _NBLK - 1)

    outp = pl.pallas_call(
        _pooltail_body,
        grid=(nblk + 1,),
        in_specs=[
            pl.BlockSpec((NC, _BR, DH), lambda i: (0, _cap(i), 0)),
            pl.BlockSpec((_BR, DH), lambda i: (_cap(i), 0)),
            pl.BlockSpec((_BR, 1), lambda i: (_cap(i), 0)),
            _full((1, DH)),
            pl.BlockSpec((1, 1, _BR), lambda i: (_cap(i), 0, 0)),
            _full((G, 4096)),
            _full((DH, 4096)),
            _full((4096, 512)),
            _full((1, 512)),
            _full((512, 128)),
            _full((1, 128)),
            _full((128, 192)),
            _full((1, 192)),
            _full((192, 8)),
            _full((1, 8)),
        ],
        out_specs=pl.BlockSpec((G, 8), lambda i: (0, 0)),
        out_shape=jax.ShapeDtypeStruct((G, 8), jnp.float32),
        scratch_shapes=[pltpu.VMEM((G, DH), jnp.float32)],
    )(z2, y2, dinv, b2p, bi3, h3p, Wg, lin2_W, lin2_b.reshape(1, 512),
      lin3_W, lin3_b.reshape(1, 128), HW1, hb1, HW2bd, hb2)
    return outp[:, :6]
